# Initial kernel scaffold; baseline (speedup 1.0000x reference)
import jax
import jax.numpy as jnp
from jax.experimental import pallas as pl


def kernel(hidden_states, q, k, v, compress_mem_kv, k_norm_w, v_norm_w, sc_w, sc_b, ch_w):
    raise NotImplementedError("write your pallas kernel here")


# speedup vs baseline: 3.8998x; 3.8998x over previous
import functools

import jax
import jax.numpy as jnp
from jax.experimental import pallas as pl


def _fused_kernel(q_ref, k_ref, v_ref, memk_ref, memv_ref, knw_ref, vnw_ref,
                  scw_ref, scb_ref, hid_ref, chw_ref, out_ref,
                  *, QB, S, HK, G, D, BS, NB, NUM_SEL, WIN, scale):
    qi = pl.program_id(0)
    base = qi * QB
    row_s = base + jax.lax.broadcasted_iota(jnp.int32, (QB, 1), 0)
    p = jax.lax.broadcasted_iota(jnp.int32, (QB, S), 1)
    colsNB = jax.lax.broadcasted_iota(jnp.int32, (QB, NB), 1)
    cmask = row_s >= (colsNB * BS + (BS - 1))

    gz = jnp.dot(hid_ref[...], scw_ref[...],
                 preferred_element_type=jnp.float32) + scb_ref[...]
    gates = jax.nn.sigmoid(gz)

    acc = jnp.zeros((QB, chw_ref.shape[1]), jnp.float32)

    for hk in range(HK):
        k = k_ref[hk]
        v = v_ref[hk]
        kb = k.reshape(NB, BS, D).mean(axis=1)
        vb = v.reshape(NB, BS, D).mean(axis=1)
        ck = kb * jax.lax.rsqrt(
            jnp.mean(kb * kb, axis=-1, keepdims=True) + 1e-6) * knw_ref[...]
        cv = vb * jax.lax.rsqrt(
            jnp.mean(vb * vb, axis=-1, keepdims=True) + 1e-6) * vnw_ref[...]
        memk = memk_ref[hk:hk + 1]
        memv = memv_ref[hk:hk + 1]

        attn_bs = []
        out_cs = []
        for g in range(G):
            qg = q_ref[hk, g]
            simc = jnp.dot(qg, ck.T, preferred_element_type=jnp.float32) * scale
            simm = jnp.dot(qg, memk.T, preferred_element_type=jnp.float32) * scale
            simc = jnp.where(cmask, simc, -1e9)
            m = jnp.maximum(jnp.max(simc, axis=-1, keepdims=True), simm)
            eb = jnp.exp(simc - m)
            em = jnp.exp(simm - m)
            denom = jnp.sum(eb, axis=-1, keepdims=True) + em
            attn_b = eb / denom
            out_c = (jnp.dot(attn_b, cv, preferred_element_type=jnp.float32)
                     + (em / denom) * memv)
            attn_bs.append(attn_b)
            out_cs.append(out_c)

        imp = (attn_bs[0] + attn_bs[1] + attn_bs[2]) * (1.0 / 3.0)
        vals = jnp.where(cmask, imp, -1.0)
        sel = []
        for _ in range(NUM_SEL):
            mx = jnp.max(vals, axis=-1, keepdims=True)
            idx = jnp.min(jnp.where(vals == mx, colsNB, NB),
                          axis=-1, keepdims=True)
            sel.append(idx)
            vals = jnp.where(colsNB == idx, -2.0, vals)

        pblock = p // BS
        selm = (pblock == sel[0]) | (pblock == sel[1]) \
            | (pblock == sel[2]) | (pblock == sel[3])
        causal = p <= row_s
        fmask = selm & causal
        rel = row_s - p
        wmask = (rel >= 0) & (rel < WIN)

        for g in range(G):
            qg = q_ref[hk, g]
            sim = jnp.dot(qg, k.T, preferred_element_type=jnp.float32) * scale
            sf = jnp.where(fmask, sim, -1e9)
            mf = jnp.max(sf, axis=-1, keepdims=True)
            ef = jnp.exp(sf - mf)
            attn_f = ef / jnp.sum(ef, axis=-1, keepdims=True)
            attn_f = jnp.where(fmask, attn_f, 0.0)
            sw = jnp.where(wmask, sim, -1e9)
            mw = jnp.max(sw, axis=-1, keepdims=True)
            ew = jnp.exp(sw - mw)
            attn_w = ew / jnp.sum(ew, axis=-1, keepdims=True)
            h = hk * G + g
            g_c = gates[:, 3 * h:3 * h + 1]
            g_f = gates[:, 3 * h + 1:3 * h + 2]
            g_w = gates[:, 3 * h + 2:3 * h + 3]
            P = g_f * attn_f + g_w * attn_w
            comb = (jnp.dot(P, v, preferred_element_type=jnp.float32)
                    + g_c * out_cs[g])
            acc = acc + jnp.dot(comb, chw_ref[h * D:(h + 1) * D, :],
                                preferred_element_type=jnp.float32)

    out_ref[...] = acc


def kernel(hidden_states, q, k, v, compress_mem_kv, k_norm_w, v_norm_w,
           sc_w, sc_b, ch_w):
    B, H, S, D = q.shape
    HK = k.shape[1]
    G = H // HK
    BS = 16
    NUM_SEL = 4
    WIN = 64
    NB = S // BS
    scale = D ** -0.5
    QB = 128
    hidden = H * D

    q4 = q[0].reshape(HK, G, S, D)
    k3 = k[0]
    v3 = v[0]
    memk = compress_mem_kv[0, :, 0, :]
    memv = compress_mem_kv[1, :, 0, :]
    knw = k_norm_w.reshape(1, D)
    vnw = v_norm_w.reshape(1, D)
    scw_t = sc_w.T
    scb = sc_b.reshape(1, 3 * H)
    chw_t = ch_w.T
    hid = hidden_states[0]

    grid = (S // QB,)
    kfn = functools.partial(_fused_kernel, QB=QB, S=S, HK=HK, G=G, D=D,
                            BS=BS, NB=NB, NUM_SEL=NUM_SEL, WIN=WIN,
                            scale=scale)
    out = pl.pallas_call(
        kfn,
        grid=grid,
        in_specs=[
            pl.BlockSpec((HK, G, QB, D), lambda i: (0, 0, i, 0)),
            pl.BlockSpec((HK, S, D), lambda i: (0, 0, 0)),
            pl.BlockSpec((HK, S, D), lambda i: (0, 0, 0)),
            pl.BlockSpec((HK, D), lambda i: (0, 0)),
            pl.BlockSpec((HK, D), lambda i: (0, 0)),
            pl.BlockSpec((1, D), lambda i: (0, 0)),
            pl.BlockSpec((1, D), lambda i: (0, 0)),
            pl.BlockSpec((hidden, 3 * H), lambda i: (0, 0)),
            pl.BlockSpec((1, 3 * H), lambda i: (0, 0)),
            pl.BlockSpec((QB, hidden), lambda i: (i, 0)),
            pl.BlockSpec((hidden, hidden), lambda i: (0, 0)),
        ],
        out_specs=pl.BlockSpec((QB, hidden), lambda i: (i, 0)),
        out_shape=jax.ShapeDtypeStruct((S, hidden), jnp.float32),
    )(q4, k3, v3, memk, memv, knw, vnw, scw_t, scb, hid, chw_t)
    return out.reshape(B, S, hidden)
